# baseline (speedup 1.0000x reference)
import jax
import jax.numpy as jnp
from jax.experimental import pallas as pl
from jax.experimental.pallas import tpu as pltpu


def _gcn_all_kernel(ts_hbm, adj_hbm, w1_ref, b1_ref, w2_ref, b2_ref,
                    wl1_ref, bl1_ref, wl2_ref, bl2_ref, out_ref,
                    ts_v, adj_v, s0, s1, s2, s3):
    B = ts_v.shape[0]
    half = B // 2
    c0 = pltpu.make_async_copy(ts_hbm.at[pl.ds(0, half)], ts_v.at[pl.ds(0, half)], s0)
    c1 = pltpu.make_async_copy(ts_hbm.at[pl.ds(half, half)], ts_v.at[pl.ds(half, half)], s1)
    c2 = pltpu.make_async_copy(adj_hbm.at[pl.ds(0, half)], adj_v.at[pl.ds(0, half)], s2)
    c3 = pltpu.make_async_copy(adj_hbm.at[pl.ds(half, half)], adj_v.at[pl.ds(half, half)], s3)
    c0.start(); c1.start(); c2.start(); c3.start()
    c0.wait(); c1.wait(); c2.wait(); c3.wait()
    adj = adj_v[...]
    ts = ts_v[...]
    B, N, _ = adj.shape

    S = jnp.sum(adj, axis=2)
    a0 = adj[0]
    cidx = jax.lax.broadcasted_iota(jnp.int32, (N, N), 1)
    c_last = jnp.max(jnp.where(a0 != 0, cidx, -1), axis=1)
    picked = jnp.sum(a0 * (cidx == c_last[:, None]), axis=1)
    loop_w = jnp.where(c_last >= 0, picked, 1.0)
    deg = jnp.sum(S[1:], axis=0) + loop_w
    deg_safe = jnp.where(deg > 0, deg, 1.0)
    dis = jnp.where(deg > 0, jax.lax.rsqrt(deg_safe), 0.0)
    coef = jnp.concatenate([(dis * dis * loop_w)[None, :],
                            dis[None, :] * S[1:]], axis=0)

    w1 = w1_ref[...]
    xw1 = jax.lax.dot_general(ts, w1, (((2,), (0,)), ((), ())),
                              preferred_element_type=jnp.float32)
    agg0 = jnp.sum(coef[:, :, None] * xw1, axis=0)
    h1 = jnp.concatenate([agg0[None], xw1[1:]], axis=0) + b1_ref[...][None, None]
    h1 = jnp.maximum(h1, 0.0)

    w2 = w2_ref[...]
    xw2 = jax.lax.dot_general(h1, w2, (((2,), (0,)), ((), ())),
                              preferred_element_type=jnp.float32)
    agg0b = jnp.sum(coef[:, :, None] * xw2, axis=0)
    h2 = jnp.concatenate([agg0b[None], xw2[1:]], axis=0) + b2_ref[...][None, None]

    p = jnp.max(h2, axis=1)

    z = jnp.maximum(
        jnp.dot(p, wl1_ref[...], preferred_element_type=jnp.float32)
        + bl1_ref[...][None], 0.0)
    out_ref[...] = (jnp.dot(z, wl2_ref[...], preferred_element_type=jnp.float32)
                    + bl2_ref[...][None])


def kernel(time_seires, node_features, W1, b1, W2, b2, Wl1, bl1, Wl2, bl2):
    B = node_features.shape[0]
    out_ch = Wl2.shape[1]
    N = node_features.shape[1]
    any_spec = pl.BlockSpec(memory_space=pltpu.MemorySpace.HBM)
    vmem_spec = pl.BlockSpec(memory_space=pltpu.MemorySpace.VMEM)
    return pl.pallas_call(
        _gcn_all_kernel,
        out_shape=jax.ShapeDtypeStruct((B, out_ch), jnp.float32),
        in_specs=[any_spec, any_spec] + [vmem_spec] * 8,
        scratch_shapes=[
            pltpu.VMEM((B, N, N), jnp.float32),
            pltpu.VMEM((B, N, N), jnp.float32),
            pltpu.SemaphoreType.DMA,
            pltpu.SemaphoreType.DMA,
            pltpu.SemaphoreType.DMA,
            pltpu.SemaphoreType.DMA,
        ],
    )(time_seires, node_features, W1, b1, W2, b2, Wl1, bl1, Wl2, bl2)


# speedup vs baseline: 1.0962x; 1.0962x over previous
import jax
import jax.numpy as jnp
from jax.experimental import pallas as pl


def _gcn_all_kernel(ts_ref, adj_ref, w1_ref, b1_ref, w2_ref, b2_ref,
                    wl1_ref, bl1_ref, wl2_ref, bl2_ref, out_ref):
    adj = adj_ref[...]
    ts = ts_ref[...]
    B, N, _ = adj.shape

    S = jnp.sum(adj, axis=2, keepdims=True)
    a0 = adj[0]
    cidx = jax.lax.broadcasted_iota(jnp.int32, (N, N), 1)
    c_last = jnp.max(jnp.where(a0 != 0, cidx, -1), axis=1, keepdims=True)
    picked = jnp.sum(a0 * (cidx == c_last), axis=1, keepdims=True)
    loop_w = jnp.where(c_last >= 0, picked, 1.0)
    deg = jnp.sum(S[1:], axis=0) + loop_w
    deg_safe = jnp.where(deg > 0, deg, 1.0)
    dis = jnp.where(deg > 0, jax.lax.rsqrt(deg_safe), 0.0)
    coef = jnp.concatenate([(dis * dis * loop_w)[None], dis[None] * S[1:]],
                           axis=0)

    xw1 = jnp.dot(ts.reshape(B * N, N), w1_ref[...],
                  preferred_element_type=jnp.float32)
    H = xw1.shape[1]
    xw1r = xw1.reshape(B, N, H)
    agg0 = jnp.sum(coef * xw1r, axis=0)
    h1 = jnp.concatenate([agg0[None], xw1r[1:]], axis=0) + b1_ref[...][None, None]
    h1 = jnp.maximum(h1, 0.0)

    xw2 = jnp.dot(h1.reshape(B * N, H), w2_ref[...],
                  preferred_element_type=jnp.float32)
    xw2r = xw2.reshape(B, N, H)
    agg0b = jnp.sum(coef * xw2r, axis=0)
    h2 = jnp.concatenate([agg0b[None], xw2r[1:]], axis=0) + b2_ref[...][None, None]

    p = jnp.max(h2, axis=1)

    z = jnp.maximum(
        jnp.dot(p, wl1_ref[...], preferred_element_type=jnp.float32)
        + bl1_ref[...][None], 0.0)
    out_ref[...] = (jnp.dot(z, wl2_ref[...], preferred_element_type=jnp.float32)
                    + bl2_ref[...][None])


def kernel(time_seires, node_features, W1, b1, W2, b2, Wl1, bl1, Wl2, bl2):
    B = node_features.shape[0]
    out_ch = Wl2.shape[1]
    return pl.pallas_call(
        _gcn_all_kernel,
        out_shape=jax.ShapeDtypeStruct((B, out_ch), jnp.float32),
    )(time_seires, node_features, W1, b1, W2, b2, Wl1, bl1, Wl2, bl2)
